# baseline (speedup 1.0000x reference)
import functools

import jax
import jax.numpy as jnp
from jax import lax
from jax.experimental import pallas as pl
from jax.experimental.pallas import tpu as pltpu
from jax.experimental.pallas import tpu_sc as plsc

MAX_N = 118
D = 128
BATCH = 16384
SEQ = 200
N = BATCH * SEQ
NC = 2
NS = 16
NW = NC * NS
BPW = N // NW
SUB = 128
CHUNK = 256
NSUB = CHUNK // SUB
NBUF = 3
ITERS = BPW // CHUNK
MAIN = (ITERS // NBUF) * NBUF
IDXR_PW = BPW // SUB


def _sc_gather(table, idx2d):
  mesh = plsc.VectorSubcoreMesh(core_axis_name="c", subcore_axis_name="s")

  @functools.partial(
      pl.kernel,
      mesh=mesh,
      out_type=jax.ShapeDtypeStruct((N, D), jnp.float32),
      scratch_types=[
          pltpu.VMEM((NBUF, NSUB, SUB), jnp.int32),
          pltpu.VMEM((NBUF, CHUNK, D), jnp.float32),
          pltpu.VMEM_SHARED((NS * 128, D), jnp.float32),
      ] + [pltpu.SemaphoreType.DMA] * (3 * NBUF),
  )
  def body(table_hbm, idx_hbm, out_hbm, idx_v, rows_v, tab_sp, *sems):
    sid = lax.axis_index("s")
    wid = sid * NC + lax.axis_index("c")
    row0 = wid * BPW
    irow0 = wid * IDXR_PW
    s_idx = sems[:NBUF]
    s_gat = sems[NBUF:2 * NBUF]
    s_out = sems[2 * NBUF:]

    pltpu.sync_copy(table_hbm, tab_sp.at[pl.ds(sid * 128, 128)])
    plsc.subcore_barrier()

    def idx_cp(i, b):
      return pltpu.make_async_copy(
          idx_hbm.at[pl.ds(irow0 + i * NSUB, NSUB)], idx_v.at[b], s_idx[b])

    def gather_cp(b, j):
      return pltpu.make_async_copy(
          tab_sp.at[idx_v.at[b].at[j]],
          rows_v.at[b].at[pl.ds(j * SUB, SUB)],
          s_gat[b])

    def out_cp(i, b):
      return pltpu.make_async_copy(
          rows_v.at[b], out_hbm.at[pl.ds(row0 + i * CHUNK, CHUNK)], s_out[b])

    def run_chunk(i, b, first, off):
      idx_cp(i, b).wait()
      for j in range(NSUB):
        for l in range(SUB // 16):
          sl = idx_v.at[b].at[j]
          sl[pl.ds(l * 16, 16)] = sl[pl.ds(l * 16, 16)] + off

      @pl.when(jnp.logical_not(first))
      def _wait_out():
        out_cp(i - NBUF, b).wait()

      for j in range(NSUB):
        gather_cp(b, j).start()
      for j in range(NSUB):
        gather_cp(b, j).wait()

      @pl.when(i + NBUF < ITERS)
      def _prefetch_idx():
        idx_cp(i + NBUF, b).start()

      out_cp(i, b).start()

    for b in range(NBUF):
      idx_cp(b, b).start()

    def step(k, carry):
      g = NBUF * k
      for b in range(NBUF):
        run_chunk(g + b, b, k < 1, sid * 128)
      return carry

    lax.fori_loop(0, MAIN // NBUF, step, 0)

    for i in range(MAIN, ITERS):
      run_chunk(i, i % NBUF, False, sid * 128)
    for i in range(ITERS - NBUF, ITERS):
      out_cp(i, i % NBUF).wait()

  return body(table, idx2d)


def kernel(inputs, embedding):
  table = jnp.zeros((128, D), jnp.float32).at[1:MAX_N + 1].set(embedding)
  idx2d = inputs.reshape(N // SUB, SUB)
  out = _sc_gather(table, idx2d)
  return out.reshape(BATCH, SEQ, D)
